# baseline (speedup 1.0000x reference)
import functools

import jax
import jax.numpy as jnp
from jax import lax
from jax.experimental import pallas as pl
from jax.experimental.pallas import tpu as pltpu
from jax.experimental.pallas import tpu_sc as plsc

_B = 16384
_D = 64
_NW = 32
_ROWS = _B // _NW
_LANES = 16
_CPD = _D // _LANES
_ICH = 128
_NICH = _ROWS // _ICH
_GRP = 8


def _make_sc_kernel():
    mesh = plsc.VectorSubcoreMesh(core_axis_name="c", subcore_axis_name="s")

    @functools.partial(
        pl.kernel,
        mesh=mesh,
        out_type=jax.ShapeDtypeStruct((_NW * _LANES,), jnp.float32),
        scratch_types=[
            pltpu.VMEM((_NICH, _ICH), jnp.int32),
            pltpu.VMEM((2, _ICH, 2 * _D), jnp.float32),
            pltpu.VMEM((_ROWS // 2, 2 * _D), jnp.float32),
            pltpu.VMEM((_ROWS // _GRP, 2 * _D), jnp.float32),
            pltpu.VMEM((_LANES,), jnp.float32),
            pltpu.SemaphoreType.DMA,
            pltpu.SemaphoreType.DMA,
            pltpu.SemaphoreType.DMA,
        ],
    )
    def sc_kernel(table_hbm, idx_hbm, w0_hbm, feats_hbm, out_hbm,
                  idx_v, rows_v, feats_v, w0_v, acc_v,
                  sem_g0, sem_g1, sem_l):
        wid = lax.axis_index("s") * 2 + lax.axis_index("c")
        sems_g = (sem_g0, sem_g1)

        pltpu.sync_copy(idx_hbm.at[pl.ds(wid * _NICH, _NICH)], idx_v)
        fcp = pltpu.async_copy(
            feats_hbm.at[pl.ds(wid * (_ROWS // 2), _ROWS // 2)],
            feats_v, sem_l)
        wcp = pltpu.async_copy(
            w0_hbm.at[pl.ds(wid * (_ROWS // _GRP), _ROWS // _GRP)],
            w0_v, sem_l)

        def fire(ch):
            return pltpu.async_copy(
                table_hbm.at[idx_v.at[ch]],
                rows_v.at[ch % 2],
                sems_g[ch % 2],
            )

        gathers = [fire(0)]
        fcp.wait()
        wcp.wait()

        zero = jnp.zeros((_LANES,), jnp.float32)
        accs = (zero,) * _CPD
        for ch in range(_NICH):
            if ch + 1 < _NICH:
                gathers.append(fire(ch + 1))
            gathers[ch].wait()
            p = ch % 2

            def body(j, accs, p=p, ch=ch):
                wrow = ch * (_ICH // _GRP) + j
                frow = ch * (_ICH // 2) + 4 * j
                out = list(accs)
                for r in range(_GRP):
                    w = w0_v[wrow, pl.ds(r * _LANES, _LANES)]
                    one_m_w = 1.0 - w
                    for c in range(_CPD):
                        f = feats_v[frow + r // 2,
                                    pl.ds((r % 2) * _D + c * _LANES, _LANES)]
                        r0 = rows_v[p, _GRP * j + r,
                                    pl.ds(c * _LANES, _LANES)]
                        r1 = rows_v[p, _GRP * j + r,
                                    pl.ds(_D + c * _LANES, _LANES)]
                        d0 = f - r0
                        d1 = f - r1
                        out[c] = (out[c] + w * (d0 * d0)
                                  + one_m_w * (d1 * d1))
                return tuple(out)

            accs = lax.fori_loop(0, _ICH // _GRP, body, accs)

        acc_v[...] = (accs[0] + accs[1]) + (accs[2] + accs[3])
        pltpu.sync_copy(acc_v, out_hbm.at[pl.ds(wid * _LANES, _LANES)])

    return sc_kernel


_SC_KERNEL = None


def kernel(feats, labels, centers):
    global _SC_KERNEL
    if _SC_KERNEL is None:
        _SC_KERNEL = _make_sc_kernel()
    labels32 = labels.astype(jnp.int32)
    table128 = centers.reshape(centers.shape[0] // 2, 2 * _D)
    idx = (labels32 >> 1).reshape(_NW * _NICH, _ICH)
    w0 = jnp.broadcast_to(
        (1 - (labels32 & 1)).astype(jnp.float32)[:, None], (_B, _LANES))
    w0 = w0.reshape(_B // _GRP, _GRP * _LANES)
    feats128 = feats.reshape(_B // 2, 2 * _D)
    partials = _SC_KERNEL(table128, idx, w0, feats128)
    return jnp.sum(partials) * (0.5 / _B)
